# Initial kernel scaffold; baseline (speedup 1.0000x reference)
import jax
import jax.numpy as jnp
from jax.experimental import pallas as pl


def kernel(x, edge_index, W1, b1, W2, b2, W3, b3):
    raise NotImplementedError("write your pallas kernel here")


# speedup vs baseline: 9.2305x; 9.2305x over previous
import functools

import jax
import jax.numpy as jnp
from jax import lax
from jax.experimental import pallas as pl
from jax.experimental.pallas import tpu as pltpu
from jax.experimental.pallas import tpu_sc as plsc

N_NODES = 10000
N_PAD = 10240
PAD_ROW = 10000
E = 160000
LANES = 16
NS = 16
NC = 2
EB = 128
NB16 = 79
NB32 = 40
ROWS_PER_TILE = N_PAD // NS
BN = 256

_mesh = plsc.VectorSubcoreMesh(core_axis_name="c", subcore_axis_name="s")


@functools.partial(
    pl.kernel,
    out_type=jax.ShapeDtypeStruct((NC, N_PAD), jnp.float32),
    mesh=_mesh,
    scratch_types=[
        pltpu.VMEM((NB32, EB), jnp.int32),
        pltpu.VMEM((EB,), jnp.float32),
        pltpu.VMEM((ROWS_PER_TILE,), jnp.float32),
        pltpu.VMEM_SHARED((N_PAD,), jnp.float32),
    ],
)
def _deg_kernel(dst_hbm, out_hbm, idx_v, ones_v, zeros_v, acc):
    c = lax.axis_index("c")
    s = lax.axis_index("s")
    w = c * NS + s

    def fill(i, carry):
        ones_v[pl.ds(i * LANES, LANES)] = jnp.full((LANES,), 1.0, jnp.float32)
        return carry
    lax.fori_loop(0, EB // LANES, fill, 0)

    def fillz(i, carry):
        zeros_v[pl.ds(i * LANES, LANES)] = jnp.zeros((LANES,), jnp.float32)
        return carry
    lax.fori_loop(0, ROWS_PER_TILE // LANES, fillz, 0)

    pltpu.sync_copy(zeros_v, acc.at[pl.ds(s * ROWS_PER_TILE, ROWS_PER_TILE)])
    pltpu.sync_copy(dst_hbm.at[w], idx_v)
    plsc.subcore_barrier()

    def body(j, carry):
        pltpu.sync_copy(ones_v, acc.at[idx_v.at[j]], add=True)
        return carry
    lax.fori_loop(0, NB32, body, 0)
    plsc.subcore_barrier()

    sl = pl.ds(s * ROWS_PER_TILE, ROWS_PER_TILE)

    @pl.when(c == 0)
    def _():
        pltpu.sync_copy(acc.at[sl], out_hbm.at[0].at[sl])

    @pl.when(c == 1)
    def _():
        pltpu.sync_copy(acc.at[sl], out_hbm.at[1].at[sl])


def _make_agg(n_chunks):
    phases = n_chunks // NC

    @functools.partial(
        pl.kernel,
        out_type=jax.ShapeDtypeStruct((n_chunks, N_PAD, 128), jnp.float32),
        mesh=_mesh,
        scratch_types=[
            pltpu.VMEM((NB16, EB), jnp.int32),
            pltpu.VMEM((NB16, EB), jnp.int32),
            pltpu.VMEM((EB, 128), jnp.float32),
            pltpu.VMEM((64, 128), jnp.float32),
            pltpu.VMEM_SHARED((N_PAD, 128), jnp.float32),
        ],
    )
    def agg(h_hbm, src_hbm, dst_hbm, out_hbm, sidx_v, didx_v, rows_v, zrow_v, acc):
        c = lax.axis_index("c")
        s = lax.axis_index("s")

        pltpu.sync_copy(src_hbm.at[s], sidx_v)
        pltpu.sync_copy(dst_hbm.at[s], didx_v)

        def fillz(k, carry):
            zrow_v[k // 8, pl.ds((k % 8) * LANES, LANES)] = jnp.zeros(
                (LANES,), jnp.float32)
            return carry
        lax.fori_loop(0, 64 * 8, fillz, 0)

        for p in range(phases):
            chunk = p * NC + c
            for k in range(ROWS_PER_TILE // 64):
                pltpu.sync_copy(zrow_v, acc.at[pl.ds(s * ROWS_PER_TILE + k * 64, 64)])
            plsc.subcore_barrier()

            def body(j, carry):
                pltpu.sync_copy(h_hbm.at[chunk].at[sidx_v.at[j]], rows_v)
                pltpu.sync_copy(rows_v, acc.at[didx_v.at[j]], add=True)
                return carry
            lax.fori_loop(0, NB16, body, 0)
            plsc.subcore_barrier()

            sl = pl.ds(s * ROWS_PER_TILE, ROWS_PER_TILE)
            pltpu.sync_copy(acc.at[sl], out_hbm.at[chunk].at[sl])
            if p + 1 < phases:
                plsc.subcore_barrier()

    return agg

_agg2 = _make_agg(2)
_agg4 = _make_agg(4)


def _prep_body(p_ref, x_ref, dis_ref, xs_ref):
    deg = p_ref[0] + p_ref[1] + 1.0
    dis = lax.rsqrt(deg)
    dis_ref[...] = dis
    prod = x_ref[...] * dis
    for c in range(2):
        xs_ref[c] = prod[:, c * 128:(c + 1) * 128]


def _prep(p3, x_pad):
    grid = N_PAD // BN
    return pl.pallas_call(
        _prep_body,
        grid=(grid,),
        in_specs=[
            pl.BlockSpec((2, BN, 1), lambda i: (0, i, 0)),
            pl.BlockSpec((BN, 256), lambda i: (i, 0)),
        ],
        out_specs=[
            pl.BlockSpec((BN, 1), lambda i: (i, 0)),
            pl.BlockSpec((2, BN, 128), lambda i: (0, i, 0)),
        ],
        out_shape=[
            jax.ShapeDtypeStruct((N_PAD, 1), jnp.float32),
            jax.ShapeDtypeStruct((2, N_PAD, 128), jnp.float32),
        ],
    )(p3, x_pad)


def _l12_body(a_ref, xs_ref, dis_ref, w1_ref, b1_ref, w2_ref, out_ref):
    dis = dis_ref[...]
    acc = b1_ref[...].astype(jnp.float32)
    for c in range(2):
        t = dis * (a_ref[c] + xs_ref[c])
        acc = acc + jnp.dot(t, w1_ref[c], preferred_element_type=jnp.float32)
    h1 = jnp.maximum(acc, 0.0)
    z2 = jnp.dot(h1, w2_ref[...], preferred_element_type=jnp.float32)
    for c in range(4):
        out_ref[c] = dis * z2[:, c * 128:(c + 1) * 128]


def _l12(a1, xs, dis, w1r, b1r, w2):
    grid = N_PAD // BN
    return pl.pallas_call(
        _l12_body,
        grid=(grid,),
        in_specs=[
            pl.BlockSpec((2, BN, 128), lambda i: (0, i, 0)),
            pl.BlockSpec((2, BN, 128), lambda i: (0, i, 0)),
            pl.BlockSpec((BN, 1), lambda i: (i, 0)),
            pl.BlockSpec((2, 128, 1024), lambda i: (0, 0, 0)),
            pl.BlockSpec((1, 1024), lambda i: (0, 0)),
            pl.BlockSpec((1024, 512), lambda i: (0, 0)),
        ],
        out_specs=pl.BlockSpec((4, BN, 128), lambda i: (0, i, 0)),
        out_shape=jax.ShapeDtypeStruct((4, N_PAD, 128), jnp.float32),
    )(a1, xs, dis, w1r, b1r, w2)


def _l23_body(s2_ref, z2_ref, dis_ref, b2_ref, w3_ref, out_ref):
    dis = dis_ref[...]
    z3 = jnp.zeros((BN, 256), jnp.float32)
    for c in range(4):
        u = dis * (s2_ref[c] + z2_ref[c]) + b2_ref[0, c * 128:(c + 1) * 128]
        h2 = jnp.maximum(u, 0.0)
        z3 = z3 + jnp.dot(h2, w3_ref[c], preferred_element_type=jnp.float32)
    for c in range(2):
        out_ref[c] = dis * z3[:, c * 128:(c + 1) * 128]


def _l23(s2, z2s, dis, b2r, w3r):
    grid = N_PAD // BN
    return pl.pallas_call(
        _l23_body,
        grid=(grid,),
        in_specs=[
            pl.BlockSpec((4, BN, 128), lambda i: (0, i, 0)),
            pl.BlockSpec((4, BN, 128), lambda i: (0, i, 0)),
            pl.BlockSpec((BN, 1), lambda i: (i, 0)),
            pl.BlockSpec((1, 512), lambda i: (0, 0)),
            pl.BlockSpec((4, 128, 256), lambda i: (0, 0, 0)),
        ],
        out_specs=pl.BlockSpec((2, BN, 128), lambda i: (0, i, 0)),
        out_shape=jax.ShapeDtypeStruct((2, N_PAD, 128), jnp.float32),
    )(s2, z2s, dis, b2r, w3r)


def _final_body(s3_ref, z3_ref, dis_ref, b3_ref, out_ref):
    dis = dis_ref[...]
    for c in range(2):
        out_ref[:, c * 128:(c + 1) * 128] = (
            dis * (s3_ref[c] + z3_ref[c]) + b3_ref[0, c * 128:(c + 1) * 128])


def _final(s3, z3s, dis, b3r):
    grid = N_PAD // BN
    return pl.pallas_call(
        _final_body,
        grid=(grid,),
        in_specs=[
            pl.BlockSpec((2, BN, 128), lambda i: (0, i, 0)),
            pl.BlockSpec((2, BN, 128), lambda i: (0, i, 0)),
            pl.BlockSpec((BN, 1), lambda i: (i, 0)),
            pl.BlockSpec((1, 256), lambda i: (0, 0)),
        ],
        out_specs=pl.BlockSpec((BN, 256), lambda i: (i, 0)),
        out_shape=jax.ShapeDtypeStruct((N_PAD, 256), jnp.float32),
    )(s3, z3s, dis, b3r)


def kernel(x, edge_index, W1, b1, W2, b2, W3, b3):
    src = edge_index[0].astype(jnp.int32)
    dst = edge_index[1].astype(jnp.int32)

    def _split(a, nway, nb):
        per = E // nway
        a = a.reshape(nway, per)
        a = jnp.pad(a, ((0, 0), (0, nb * EB - per)), constant_values=PAD_ROW)
        return a.reshape(nway, nb, EB)

    src16 = _split(src, NS, NB16)
    dst16 = _split(dst, NS, NB16)
    dst32 = _split(dst, NC * NS, NB32)

    x_pad = jnp.pad(x, ((0, N_PAD - N_NODES), (0, 0)))
    w1r = W1.reshape(2, 128, 1024)
    b1r = b1.reshape(1, 1024)
    w2 = W2
    b2r = b2.reshape(1, 512)
    w3r = W3.reshape(4, 128, 256)
    b3r = b3.reshape(1, 256)

    p = _deg_kernel(dst32)
    p3 = p.reshape(2, N_PAD, 1)
    dis, xs = _prep(p3, x_pad)
    a1 = _agg2(xs, src16, dst16)
    z2s = _l12(a1, xs, dis, w1r, b1r, w2)
    s2 = _agg4(z2s, src16, dst16)
    z3s = _l23(s2, z2s, dis, b2r, w3r)
    s3 = _agg2(z3s, src16, dst16)
    out = _final(s3, z3s, dis, b3r)
    return out[:N_NODES]
